# baseline (speedup 1.0000x reference)
import functools
import jax
import jax.numpy as jnp
from jax import lax
from jax.experimental import pallas as pl
from jax.experimental.pallas import tpu as pltpu
from jax.experimental.pallas import tpu_sc as plsc


def kernel(x, spatial_emb):
    b, c, h, w = x.shape
    g = spatial_emb.shape[1]
    ch, cw = h // g, w // g
    band = ch * w
    k = 1
    while (k * band) % 128:
        k += 1
    lanes = k * band
    nrg = g // k
    kg = k * g
    rows_total = b * c * nrg
    tab_rows = c * nrg
    nchunk = lanes // 16

    info = plsc.get_sparse_core_info()
    nc, ns = info.num_cores, info.num_subcores
    nw = nc * ns
    rpw = rows_total // nw

    tab = jnp.transpose(spatial_emb[0], (2, 0, 1)).reshape(tab_rows * kg)
    l = jnp.arange(lanes, dtype=jnp.int32)
    code = (l // band) * g + (l % w) // cw
    x_flat = x.reshape(rows_total * lanes)

    mesh = plsc.VectorSubcoreMesh(core_axis_name="c", subcore_axis_name="s")

    @functools.partial(
        pl.kernel,
        out_type=jax.ShapeDtypeStruct((rows_total * lanes,), jnp.float32),
        mesh=mesh,
        scratch_types=[
            pltpu.VMEM((rpw * kg,), jnp.float32),
            pltpu.VMEM((lanes,), jnp.int32),
            pltpu.VMEM((lanes,), jnp.float32),
        ],
        compiler_params=pltpu.CompilerParams(needs_layout_passes=False),
    )
    def sc_add(x_hbm, tab_hbm, code_hbm, out_hbm, tab_v, code_v, buf_v):
        wid = lax.axis_index("s") * nc + lax.axis_index("c")
        pltpu.sync_copy(code_hbm, code_v)
        lrow0 = lax.rem(wid * rpw, tab_rows)
        pltpu.sync_copy(tab_hbm.at[pl.ds(lrow0 * kg, rpw * kg)], tab_v)

        def row_body(i, carry):
            r = wid * rpw + i
            pltpu.sync_copy(x_hbm.at[pl.ds(r * lanes, lanes)], buf_v)

            def chunk_body(j, carry2):
                off = j * 16
                idx = code_v[pl.ds(off, 16)] + i * kg
                ev = plsc.load_gather(tab_v, [idx])
                buf_v[pl.ds(off, 16)] = buf_v[pl.ds(off, 16)] + ev
                return carry2

            lax.fori_loop(0, nchunk, chunk_body, 0)
            pltpu.sync_copy(buf_v, out_hbm.at[pl.ds(r * lanes, lanes)])
            return carry

        lax.fori_loop(0, rpw, row_body, 0)

    out = sc_add(x_flat, tab, code)
    return out.reshape(b, c, h, w)


# speedup vs baseline: 1.9980x; 1.9980x over previous
import functools
import jax
import jax.numpy as jnp
from jax import lax
from jax.experimental import pallas as pl
from jax.experimental.pallas import tpu as pltpu
from jax.experimental.pallas import tpu_sc as plsc


def kernel(x, spatial_emb):
    b, c, h, w = x.shape
    g = spatial_emb.shape[1]
    ch, cw = h // g, w // g
    band = ch * w
    k = 1
    while (k * band) % 128:
        k += 1
    lanes = k * band
    nrg = g // k
    kg = k * g
    tab_rows = c * nrg
    nchunk = lanes // 16

    info = plsc.get_sparse_core_info()
    nc, ns = info.num_cores, info.num_subcores
    nw = nc * ns
    ppw = tab_rows // nw
    CH = 2
    nck = ppw // CH
    chl = CH * lanes

    tab = jnp.transpose(spatial_emb[0], (2, 0, 1)).reshape(tab_rows * kg)
    l = jnp.arange(lanes, dtype=jnp.int32)
    code = (l // band) * g + (l % w) // cw
    x_flat = x.reshape(b * tab_rows * lanes)
    img = tab_rows * lanes

    mesh = plsc.VectorSubcoreMesh(core_axis_name="c", subcore_axis_name="s")

    @functools.partial(
        pl.kernel,
        out_type=jax.ShapeDtypeStruct((b * tab_rows * lanes,), jnp.float32),
        mesh=mesh,
        scratch_types=(
            [pltpu.VMEM((ppw * kg,), jnp.float32),
             pltpu.VMEM((lanes,), jnp.int32)]
            + [pltpu.VMEM((chl,), jnp.float32) for _ in range(4)]
            + [pltpu.VMEM((chl,), jnp.float32) for _ in range(4)]
            + [pltpu.SemaphoreType.DMA for _ in range(8)]
        ),
        compiler_params=pltpu.CompilerParams(needs_layout_passes=False),
    )
    def sc_add(x_hbm, tab_hbm, code_hbm, out_hbm,
               tab_v, code_v,
               xa0, xa1, xb0, xb1, oa0, oa1, ob0, ob1,
               sia0, sia1, sib0, sib1, soa0, soa1, sob0, sob1):
        wid = lax.axis_index("s") * nc + lax.axis_index("c")
        pltpu.sync_copy(code_hbm, code_v)
        pltpu.sync_copy(tab_hbm.at[pl.ds(wid * (ppw * kg), ppw * kg)], tab_v)
        base = wid * ppw * lanes

        xin = [[xa0, xb0], [xa1, xb1]]
        obuf = [[oa0, ob0], [oa1, ob1]]
        sin = [[sia0, sib0], [sia1, sib1]]
        sout = [[soa0, sob0], [soa1, sob1]]

        def in_copy(gi, s):
            for t in range(2):
                src = x_hbm.at[pl.ds(t * img + base + gi * chl, chl)]
                yield pltpu.make_async_copy(src, xin[s][t], sin[s][t])

        def out_copy(gi, s):
            for t in range(2):
                dst = out_hbm.at[pl.ds(t * img + base + gi * chl, chl)]
                yield pltpu.make_async_copy(obuf[s][t], dst, sout[s][t])

        for s in range(2):
            for cp in in_copy(s, s):
                cp.start()

        for gi in range(nck):
            s = gi & 1
            for cp in in_copy(gi, s):
                cp.wait()
            if gi >= 2:
                for cp in out_copy(gi - 2, s):
                    cp.wait()
            xa, xb = xin[s]
            oa, ob = obuf[s]
            for rin in range(CH):
                roff = rin * lanes
                ibase = (gi * CH + rin) * kg

                @plsc.parallel_loop(0, nchunk, unroll=4)
                def _(j):
                    off = j * 16
                    idx = code_v[pl.ds(off, 16)] + ibase
                    ev = plsc.load_gather(tab_v, [idx])
                    oa[pl.ds(roff + off, 16)] = xa[pl.ds(roff + off, 16)] + ev
                    ob[pl.ds(roff + off, 16)] = xb[pl.ds(roff + off, 16)] + ev

            for cp in out_copy(gi, s):
                cp.start()
            if gi + 2 < nck:
                for cp in in_copy(gi + 2, s):
                    cp.start()

        for gi in (nck - 2, nck - 1):
            for cp in out_copy(gi, gi & 1):
                cp.wait()

    out = sc_add(x_flat, tab, code)
    return out.reshape(b, c, h, w)
